# baseline (speedup 1.0000x reference)
import functools

import jax
import jax.numpy as jnp
from jax import lax
from jax.experimental import pallas as pl
from jax.experimental.pallas import tpu as pltpu
from jax.experimental.pallas import tpu_sc as plsc

N_POINTS = 320000
N_VOXELS = 10000
IN_CH = 128
OUT_CH = 128
HID = 4 * OUT_CH


ROWS_PER_BLOCK = 4000

_DN_T = (((1,), (1,)), ((), ()))


def _fold_body(wl_ref, w1_ref, o_ref):
    o_ref[...] = jax.lax.dot_general(
        wl_ref[...], w1_ref[...], (((0,), (1,)), ((), ())),
        preferred_element_type=jnp.float32,
    ).astype(jnp.bfloat16)


def _fold_w1(W_lin, W1):
    return pl.pallas_call(
        _fold_body,
        out_shape=jax.ShapeDtypeStruct((IN_CH, HID), jnp.bfloat16),
    )(W_lin, W1)


def _mlp_body(x_ref, wl_ref, w1l_ref, b1_ref, w2_ref, b2_ref, o_ref):
    x = x_ref[...].astype(jnp.bfloat16)
    p = jax.lax.dot_general(x, wl_ref[...].astype(jnp.bfloat16), _DN_T,
                            preferred_element_type=jnp.float32)
    h = jnp.dot(x, w1l_ref[...],
                preferred_element_type=jnp.float32).astype(jnp.bfloat16)
    h = jnp.maximum(h + b1_ref[...], jnp.bfloat16(0.0))
    a = jax.lax.dot_general(h, w2_ref[...].astype(jnp.bfloat16), _DN_T,
                            preferred_element_type=jnp.float32)
    a = jax.nn.sigmoid(a + b2_ref[...])
    o_ref[...] = p * a


def _mlp_attend(points, W_lin, w1l_bf, b1r, W2, b2r):
    n = points.shape[0]
    return pl.pallas_call(
        _mlp_body,
        grid=(n // ROWS_PER_BLOCK,),
        in_specs=[
            pl.BlockSpec((ROWS_PER_BLOCK, IN_CH), lambda i: (i, 0)),
            pl.BlockSpec((OUT_CH, IN_CH), lambda i: (0, 0)),
            pl.BlockSpec((IN_CH, HID), lambda i: (0, 0)),
            pl.BlockSpec((1, HID), lambda i: (0, 0)),
            pl.BlockSpec((OUT_CH, HID), lambda i: (0, 0)),
            pl.BlockSpec((1, OUT_CH), lambda i: (0, 0)),
        ],
        out_specs=pl.BlockSpec((ROWS_PER_BLOCK, OUT_CH), lambda i: (i, 0)),
        out_shape=jax.ShapeDtypeStruct((n, OUT_CH), jnp.float32),
    )(points, W_lin, w1l_bf, b1r.astype(jnp.bfloat16), W2, b2r)


CHUNK = 128
N_CHUNKS = N_POINTS // CHUNK
HALF = N_VOXELS // 2
ACC_ROWS = 5120
ZROWS = ACC_ROWS // 16
OUT_RC = 40
N_OUT_CHUNKS = HALF // OUT_RC

@functools.cache
def _make_segment_sum_sc():
    mesh = plsc.VectorSubcoreMesh(core_axis_name="c", subcore_axis_name="s")
    return functools.partial(
        pl.kernel,
        out_type=jax.ShapeDtypeStruct((N_VOXELS, OUT_CH), jnp.float32),
        mesh=mesh,
        scratch_types=[
            pltpu.VMEM((CHUNK,), jnp.int32),
            pltpu.VMEM((CHUNK,), jnp.int32),
            pltpu.VMEM((CHUNK,), jnp.int32),
            pltpu.VMEM((CHUNK, OUT_CH), jnp.float32),
            pltpu.VMEM((CHUNK, OUT_CH), jnp.float32),
            pltpu.VMEM((16,), jnp.int32),
            pltpu.VMEM((16,), jnp.int32),
            pltpu.VMEM((ZROWS, OUT_CH), jnp.float32),
            pltpu.VMEM_SHARED((ACC_ROWS, OUT_CH), jnp.float32),
            pltpu.SemaphoreType.DMA,
            pltpu.SemaphoreType.DMA,
        ],
    )(_segment_sum_body)


def _segment_sum_body(rows_hbm, inv_hbm, starts_hbm, ends_hbm, out_hbm,
                      idx0, idx1, loc_v, rows0, rows1, st_v, en_v, zbuf, acc,
                      sem0, sem1):
    c = lax.axis_index("c")
    s = lax.axis_index("s")

    zeros16 = jnp.zeros((16,), jnp.float32)

    def _zero_row(r, carry):
        for j in range(OUT_CH // 16):
            zbuf[r, pl.ds(j * 16, 16)] = zeros16
        return carry

    lax.fori_loop(0, ZROWS, _zero_row, 0)
    pltpu.sync_copy(zbuf, acc.at[pl.ds(s * ZROWS, ZROWS)])
    plsc.subcore_barrier()

    pltpu.sync_copy(starts_hbm.at[c, s], st_v)
    pltpu.sync_copy(ends_hbm.at[c, s], en_v)
    st = st_v[...][0]
    en = en_v[...][0]

    base = c * HALF
    bufs = ((idx0, rows0, sem0), (idx1, rows1, sem1))

    def _start(k, b):
        idx_b, rows_b, sem_b = bufs[b]
        pltpu.async_copy(inv_hbm.at[pl.ds(k * CHUNK, CHUNK)], idx_b, sem_b)
        pltpu.async_copy(rows_hbm.at[pl.ds(k * CHUNK, CHUNK)], rows_b, sem_b)

    def _wait(b):
        idx_b, rows_b, sem_b = bufs[b]
        pltpu.make_async_copy(inv_hbm.at[pl.ds(0, CHUNK)], idx_b, sem_b).wait()
        pltpu.make_async_copy(rows_hbm.at[pl.ds(0, CHUNK)], rows_b, sem_b).wait()

    def _process(b):
        idx_b, rows_b, _ = bufs[b]
        for j in range(CHUNK // 16):
            iv = idx_b[pl.ds(j * 16, 16)]
            valid = (iv >= base) & (iv < base + HALF)
            loc_v[pl.ds(j * 16, 16)] = jnp.where(valid, iv - base, HALF)
        pltpu.sync_copy(rows_b, acc.at[loc_v], add=True)

    @pl.when(st < en)
    def _prime0():
        _start(st, 0)

    @pl.when(st + 1 < en)
    def _prime1():
        _start(st + 1, 1)

    def _pair(g, carry):
        k0 = st + 2 * g
        for b in range(2):
            k = k0 + b

            @pl.when(k < en)
            def _do(k=k, b=b):
                _wait(b)
                _process(b)

                @pl.when(k + 2 < en)
                def _next():
                    _start(k + 2, b)

        return carry

    lax.fori_loop(0, (en - st + 1) // 2, _pair, 0)
    plsc.subcore_barrier()

    def _out(i, carry):
        cid = s + i * 16

        @pl.when(cid < N_OUT_CHUNKS)
        def _copy_out():
            pltpu.sync_copy(
                acc.at[pl.ds(cid * OUT_RC, OUT_RC)],
                out_hbm.at[pl.ds(c * HALF + cid * OUT_RC, OUT_RC)],
            )

        return carry

    lax.fori_loop(0, (N_OUT_CHUNKS + 15) // 16, _out, 0)


def _add_body(a_ref, b_ref, o_ref):
    o_ref[...] = a_ref[...] + b_ref[...]


def _combine(a, b):
    return pl.pallas_call(
        _add_body,
        grid=(5,),
        in_specs=[
            pl.BlockSpec((N_VOXELS // 5, OUT_CH), lambda i: (i, 0)),
            pl.BlockSpec((N_VOXELS // 5, OUT_CH), lambda i: (i, 0)),
        ],
        out_specs=pl.BlockSpec((N_VOXELS // 5, OUT_CH), lambda i: (i, 0)),
        out_shape=jax.ShapeDtypeStruct((N_VOXELS, OUT_CH), jnp.float32),
    )(a, b)


def _chunk_ranges(inv_part):
    n_chunks = inv_part.shape[0] // CHUNK
    split = jnp.sum((inv_part < HALF).astype(jnp.int32)).astype(jnp.int32)
    cs0_end = (split + CHUNK - 1) // CHUNK
    cs1_start = split // CHUNK
    w = jnp.arange(16, dtype=jnp.int32)
    starts0 = w * cs0_end // 16
    ends0 = (w + 1) * cs0_end // 16
    n1 = n_chunks - cs1_start
    starts1 = cs1_start + w * n1 // 16
    ends1 = cs1_start + (w + 1) * n1 // 16
    starts = jnp.broadcast_to(
        jnp.stack([starts0, starts1]).astype(jnp.int32)[:, :, None], (2, 16, 16)
    )
    ends = jnp.broadcast_to(
        jnp.stack([ends0, ends1]).astype(jnp.int32)[:, :, None], (2, 16, 16)
    )
    return starts, ends


def kernel(points, _inv, W_lin, W1, b1, W2, b2):
    inv32 = _inv.astype(jnp.int32)
    nh = N_POINTS // 2
    w1l = _fold_w1(W_lin, W1)
    b1r = b1.reshape(1, HID)
    b2r = b2.reshape(1, OUT_CH)
    seg = _make_segment_sum_sc()

    inv_a, inv_b = inv32[:nh], inv32[nh:]
    st_a, en_a = _chunk_ranges(inv_a)
    st_b, en_b = _chunk_ranges(inv_b)

    out_a = _mlp_attend(points[:nh], W_lin, w1l, b1r, W2, b2r)
    buf_a = seg(out_a, inv_a, st_a, en_a)
    out_b = _mlp_attend(points[nh:], W_lin, w1l, b1r, W2, b2r)
    buf_b = seg(out_b, inv_b, st_b, en_b)
    return _combine(buf_a, buf_b)


# speedup vs baseline: 1.3617x; 1.3617x over previous
import functools

import jax
import jax.numpy as jnp
from jax import lax
from jax.experimental import pallas as pl
from jax.experimental.pallas import tpu as pltpu
from jax.experimental.pallas import tpu_sc as plsc

N_POINTS = 320000
N_VOXELS = 10000
IN_CH = 128
OUT_CH = 128
HID = 4 * OUT_CH


ROWS_PER_BLOCK = 8000
N_BLOCKS = N_POINTS // ROWS_PER_BLOCK

_DN_T = (((1,), (1,)), ((), ()))


def _fold_body(wl_ref, w1_ref, o_ref):
    o_ref[...] = jax.lax.dot_general(
        wl_ref[...], w1_ref[...], (((0,), (1,)), ((), ())),
        preferred_element_type=jnp.float32,
    ).astype(jnp.bfloat16)


def _fold_w1(W_lin, W1):
    return pl.pallas_call(
        _fold_body,
        out_shape=jax.ShapeDtypeStruct((IN_CH, HID), jnp.bfloat16),
    )(W_lin, W1)


def _mlp_body(x_ref, wl_ref, w1l_ref, b1_ref, w2_ref, b2_ref, o_ref):
    x = x_ref[...].astype(jnp.bfloat16)
    p = jax.lax.dot_general(x, wl_ref[...].astype(jnp.bfloat16), _DN_T,
                            preferred_element_type=jnp.float32)
    h = jnp.dot(x, w1l_ref[...],
                preferred_element_type=jnp.float32).astype(jnp.bfloat16)
    h = jnp.maximum(h + b1_ref[...], jnp.bfloat16(0.0))
    a = jax.lax.dot_general(h, w2_ref[...].astype(jnp.bfloat16), _DN_T,
                            preferred_element_type=jnp.float32)
    a = jax.nn.sigmoid(a + b2_ref[...])
    o_ref[...] = p * a


def _mlp_attend(points, W_lin, w1l_bf, b1r, W2, b2r):
    return pl.pallas_call(
        _mlp_body,
        grid=(N_BLOCKS,),
        in_specs=[
            pl.BlockSpec((ROWS_PER_BLOCK, IN_CH), lambda i: (i, 0)),
            pl.BlockSpec((OUT_CH, IN_CH), lambda i: (0, 0)),
            pl.BlockSpec((IN_CH, HID), lambda i: (0, 0)),
            pl.BlockSpec((1, HID), lambda i: (0, 0)),
            pl.BlockSpec((OUT_CH, HID), lambda i: (0, 0)),
            pl.BlockSpec((1, OUT_CH), lambda i: (0, 0)),
        ],
        out_specs=pl.BlockSpec((ROWS_PER_BLOCK, OUT_CH), lambda i: (i, 0)),
        out_shape=jax.ShapeDtypeStruct((N_POINTS, OUT_CH), jnp.float32),
    )(points, W_lin, w1l_bf, b1r.astype(jnp.bfloat16), W2, b2r)


CHUNK = 128
N_CHUNKS = N_POINTS // CHUNK
HALF = N_VOXELS // 2
ACC_ROWS = 5120
ZROWS = ACC_ROWS // 16
OUT_RC = 40
N_OUT_CHUNKS = HALF // OUT_RC

@functools.cache
def _make_segment_sum_sc():
    mesh = plsc.VectorSubcoreMesh(core_axis_name="c", subcore_axis_name="s")
    return functools.partial(
        pl.kernel,
        out_type=jax.ShapeDtypeStruct((N_VOXELS, OUT_CH), jnp.float32),
        mesh=mesh,
        scratch_types=[
            pltpu.VMEM((CHUNK,), jnp.int32),
            pltpu.VMEM((CHUNK,), jnp.int32),
            pltpu.VMEM((CHUNK,), jnp.int32),
            pltpu.VMEM((CHUNK, OUT_CH), jnp.float32),
            pltpu.VMEM((CHUNK, OUT_CH), jnp.float32),
            pltpu.VMEM((16,), jnp.int32),
            pltpu.VMEM((16,), jnp.int32),
            pltpu.VMEM((ZROWS, OUT_CH), jnp.float32),
            pltpu.VMEM_SHARED((ACC_ROWS, OUT_CH), jnp.float32),
            pltpu.SemaphoreType.DMA,
            pltpu.SemaphoreType.DMA,
        ],
    )(_segment_sum_body)


def _segment_sum_body(rows_hbm, inv_hbm, starts_hbm, ends_hbm, out_hbm,
                      idx0, idx1, loc_v, rows0, rows1, st_v, en_v, zbuf, acc,
                      sem0, sem1):
    c = lax.axis_index("c")
    s = lax.axis_index("s")

    zeros16 = jnp.zeros((16,), jnp.float32)

    def _zero_row(r, carry):
        for j in range(OUT_CH // 16):
            zbuf[r, pl.ds(j * 16, 16)] = zeros16
        return carry

    lax.fori_loop(0, ZROWS, _zero_row, 0)
    pltpu.sync_copy(zbuf, acc.at[pl.ds(s * ZROWS, ZROWS)])
    plsc.subcore_barrier()

    pltpu.sync_copy(starts_hbm.at[c, s], st_v)
    pltpu.sync_copy(ends_hbm.at[c, s], en_v)
    st = st_v[...][0]
    en = en_v[...][0]

    base = c * HALF
    bufs = ((idx0, rows0, sem0), (idx1, rows1, sem1))

    def _start(k, b):
        idx_b, rows_b, sem_b = bufs[b]
        pltpu.async_copy(inv_hbm.at[pl.ds(k * CHUNK, CHUNK)], idx_b, sem_b)
        pltpu.async_copy(rows_hbm.at[pl.ds(k * CHUNK, CHUNK)], rows_b, sem_b)

    def _wait(b):
        idx_b, rows_b, sem_b = bufs[b]
        pltpu.make_async_copy(inv_hbm.at[pl.ds(0, CHUNK)], idx_b, sem_b).wait()
        pltpu.make_async_copy(rows_hbm.at[pl.ds(0, CHUNK)], rows_b, sem_b).wait()

    def _process(b):
        idx_b, rows_b, _ = bufs[b]
        for j in range(CHUNK // 16):
            iv = idx_b[pl.ds(j * 16, 16)]
            valid = (iv >= base) & (iv < base + HALF)
            loc_v[pl.ds(j * 16, 16)] = jnp.where(valid, iv - base, HALF)
        pltpu.sync_copy(rows_b, acc.at[loc_v], add=True)

    @pl.when(st < en)
    def _prime0():
        _start(st, 0)

    @pl.when(st + 1 < en)
    def _prime1():
        _start(st + 1, 1)

    def _pair(g, carry):
        k0 = st + 2 * g
        for b in range(2):
            k = k0 + b

            @pl.when(k < en)
            def _do(k=k, b=b):
                _wait(b)
                _process(b)

                @pl.when(k + 2 < en)
                def _next():
                    _start(k + 2, b)

        return carry

    lax.fori_loop(0, (en - st + 1) // 2, _pair, 0)
    plsc.subcore_barrier()

    def _out(i, carry):
        cid = s + i * 16

        @pl.when(cid < N_OUT_CHUNKS)
        def _copy_out():
            pltpu.sync_copy(
                acc.at[pl.ds(cid * OUT_RC, OUT_RC)],
                out_hbm.at[pl.ds(c * HALF + cid * OUT_RC, OUT_RC)],
            )

        return carry

    lax.fori_loop(0, (N_OUT_CHUNKS + 15) // 16, _out, 0)


def kernel(points, _inv, W_lin, W1, b1, W2, b2):
    inv32 = _inv.astype(jnp.int32)
    w1l = _fold_w1(W_lin, W1)
    out_pts = _mlp_attend(
        points, W_lin, w1l, b1.reshape(1, HID), W2, b2.reshape(1, OUT_CH)
    )

    split = jnp.sum((inv32 < HALF).astype(jnp.int32)).astype(jnp.int32)
    cs0_end = (split + CHUNK - 1) // CHUNK
    cs1_start = split // CHUNK
    w = jnp.arange(16, dtype=jnp.int32)
    starts0 = w * cs0_end // 16
    ends0 = (w + 1) * cs0_end // 16
    n1 = N_CHUNKS - cs1_start
    starts1 = cs1_start + w * n1 // 16
    ends1 = cs1_start + (w + 1) * n1 // 16
    starts = jnp.broadcast_to(
        jnp.stack([starts0, starts1]).astype(jnp.int32)[:, :, None], (2, 16, 16)
    )
    ends = jnp.broadcast_to(
        jnp.stack([ends0, ends1]).astype(jnp.int32)[:, :, None], (2, 16, 16)
    )

    return _make_segment_sum_sc()(out_pts, inv32, starts, ends)


# speedup vs baseline: 1.3851x; 1.0172x over previous
import functools

import jax
import jax.numpy as jnp
from jax import lax
from jax.experimental import pallas as pl
from jax.experimental.pallas import tpu as pltpu
from jax.experimental.pallas import tpu_sc as plsc

N_POINTS = 320000
N_VOXELS = 10000
IN_CH = 128
OUT_CH = 128
HID = 4 * OUT_CH


ROWS_PER_BLOCK = 16000
N_BLOCKS = N_POINTS // ROWS_PER_BLOCK

_DN_T = (((1,), (1,)), ((), ()))


def _fold_body(wl_ref, w1_ref, o_ref):
    o_ref[...] = jax.lax.dot_general(
        wl_ref[...], w1_ref[...], (((0,), (1,)), ((), ())),
        preferred_element_type=jnp.float32,
    ).astype(jnp.bfloat16)


def _fold_w1(W_lin, W1):
    return pl.pallas_call(
        _fold_body,
        out_shape=jax.ShapeDtypeStruct((IN_CH, HID), jnp.bfloat16),
    )(W_lin, W1)


def _mlp_body(x_ref, wl_ref, w1l_ref, b1_ref, w2_ref, b2_ref, o_ref):
    x = x_ref[...].astype(jnp.bfloat16)
    p = jax.lax.dot_general(x, wl_ref[...].astype(jnp.bfloat16), _DN_T,
                            preferred_element_type=jnp.float32)
    h = jnp.dot(x, w1l_ref[...],
                preferred_element_type=jnp.float32).astype(jnp.bfloat16)
    h = jnp.maximum(h + b1_ref[...], jnp.bfloat16(0.0))
    a = jax.lax.dot_general(h, w2_ref[...].astype(jnp.bfloat16), _DN_T,
                            preferred_element_type=jnp.float32)
    a = jax.nn.sigmoid(a + b2_ref[...])
    o_ref[...] = p * a


def _mlp_attend(points, W_lin, w1l_bf, b1r, W2, b2r):
    return pl.pallas_call(
        _mlp_body,
        grid=(N_BLOCKS,),
        in_specs=[
            pl.BlockSpec((ROWS_PER_BLOCK, IN_CH), lambda i: (i, 0)),
            pl.BlockSpec((OUT_CH, IN_CH), lambda i: (0, 0)),
            pl.BlockSpec((IN_CH, HID), lambda i: (0, 0)),
            pl.BlockSpec((1, HID), lambda i: (0, 0)),
            pl.BlockSpec((OUT_CH, HID), lambda i: (0, 0)),
            pl.BlockSpec((1, OUT_CH), lambda i: (0, 0)),
        ],
        out_specs=pl.BlockSpec((ROWS_PER_BLOCK, OUT_CH), lambda i: (i, 0)),
        out_shape=jax.ShapeDtypeStruct((N_POINTS, OUT_CH), jnp.float32),
    )(points, W_lin, w1l_bf, b1r.astype(jnp.bfloat16), W2, b2r)


CHUNK = 128
N_CHUNKS = N_POINTS // CHUNK
HALF = N_VOXELS // 2
ACC_ROWS = 5120
ZROWS = ACC_ROWS // 16
OUT_RC = 40
N_OUT_CHUNKS = HALF // OUT_RC

@functools.cache
def _make_segment_sum_sc():
    mesh = plsc.VectorSubcoreMesh(core_axis_name="c", subcore_axis_name="s")
    return functools.partial(
        pl.kernel,
        out_type=jax.ShapeDtypeStruct((N_VOXELS, OUT_CH), jnp.float32),
        mesh=mesh,
        scratch_types=[
            pltpu.VMEM((CHUNK,), jnp.int32),
            pltpu.VMEM((CHUNK,), jnp.int32),
            pltpu.VMEM((CHUNK,), jnp.int32),
            pltpu.VMEM((CHUNK, OUT_CH), jnp.float32),
            pltpu.VMEM((CHUNK, OUT_CH), jnp.float32),
            pltpu.VMEM((16,), jnp.int32),
            pltpu.VMEM((16,), jnp.int32),
            pltpu.VMEM((ZROWS, OUT_CH), jnp.float32),
            pltpu.VMEM_SHARED((ACC_ROWS, OUT_CH), jnp.float32),
            pltpu.SemaphoreType.DMA,
            pltpu.SemaphoreType.DMA,
        ],
    )(_segment_sum_body)


def _segment_sum_body(rows_hbm, inv_hbm, starts_hbm, ends_hbm, out_hbm,
                      idx0, idx1, loc_v, rows0, rows1, st_v, en_v, zbuf, acc,
                      sem0, sem1):
    c = lax.axis_index("c")
    s = lax.axis_index("s")

    zeros16 = jnp.zeros((16,), jnp.float32)

    def _zero_row(r, carry):
        for j in range(OUT_CH // 16):
            zbuf[r, pl.ds(j * 16, 16)] = zeros16
        return carry

    lax.fori_loop(0, ZROWS, _zero_row, 0)
    pltpu.sync_copy(zbuf, acc.at[pl.ds(s * ZROWS, ZROWS)])
    plsc.subcore_barrier()

    pltpu.sync_copy(starts_hbm.at[c, s], st_v)
    pltpu.sync_copy(ends_hbm.at[c, s], en_v)
    st = st_v[...][0]
    en = en_v[...][0]

    base = c * HALF
    bufs = ((idx0, rows0, sem0), (idx1, rows1, sem1))

    def _start(k, b):
        idx_b, rows_b, sem_b = bufs[b]
        pltpu.async_copy(inv_hbm.at[pl.ds(k * CHUNK, CHUNK)], idx_b, sem_b)
        pltpu.async_copy(rows_hbm.at[pl.ds(k * CHUNK, CHUNK)], rows_b, sem_b)

    def _wait(b):
        idx_b, rows_b, sem_b = bufs[b]
        pltpu.make_async_copy(inv_hbm.at[pl.ds(0, CHUNK)], idx_b, sem_b).wait()
        pltpu.make_async_copy(rows_hbm.at[pl.ds(0, CHUNK)], rows_b, sem_b).wait()

    def _process(b):
        idx_b, rows_b, _ = bufs[b]
        for j in range(CHUNK // 16):
            iv = idx_b[pl.ds(j * 16, 16)]
            valid = (iv >= base) & (iv < base + HALF)
            loc_v[pl.ds(j * 16, 16)] = jnp.where(valid, iv - base, HALF)
        pltpu.sync_copy(rows_b, acc.at[loc_v], add=True)

    @pl.when(st < en)
    def _prime0():
        _start(st, 0)

    @pl.when(st + 1 < en)
    def _prime1():
        _start(st + 1, 1)

    def _pair(g, carry):
        k0 = st + 2 * g
        for b in range(2):
            k = k0 + b

            @pl.when(k < en)
            def _do(k=k, b=b):
                _wait(b)
                _process(b)

                @pl.when(k + 2 < en)
                def _next():
                    _start(k + 2, b)

        return carry

    lax.fori_loop(0, (en - st + 1) // 2, _pair, 0)
    plsc.subcore_barrier()

    def _out(i, carry):
        cid = s + i * 16

        @pl.when(cid < N_OUT_CHUNKS)
        def _copy_out():
            pltpu.sync_copy(
                acc.at[pl.ds(cid * OUT_RC, OUT_RC)],
                out_hbm.at[pl.ds(c * HALF + cid * OUT_RC, OUT_RC)],
            )

        return carry

    lax.fori_loop(0, (N_OUT_CHUNKS + 15) // 16, _out, 0)


def kernel(points, _inv, W_lin, W1, b1, W2, b2):
    inv32 = _inv.astype(jnp.int32)
    w1l = _fold_w1(W_lin, W1)
    out_pts = _mlp_attend(
        points, W_lin, w1l, b1.reshape(1, HID), W2, b2.reshape(1, OUT_CH)
    )

    split = jnp.sum((inv32 < HALF).astype(jnp.int32)).astype(jnp.int32)
    cs0_end = (split + CHUNK - 1) // CHUNK
    cs1_start = split // CHUNK
    w = jnp.arange(16, dtype=jnp.int32)
    starts0 = w * cs0_end // 16
    ends0 = (w + 1) * cs0_end // 16
    n1 = N_CHUNKS - cs1_start
    starts1 = cs1_start + w * n1 // 16
    ends1 = cs1_start + (w + 1) * n1 // 16
    starts = jnp.broadcast_to(
        jnp.stack([starts0, starts1]).astype(jnp.int32)[:, :, None], (2, 16, 16)
    )
    ends = jnp.broadcast_to(
        jnp.stack([ends0, ends1]).astype(jnp.int32)[:, :, None], (2, 16, 16)
    )

    return _make_segment_sum_sc()(out_pts, inv32, starts, ends)
